# baseline (speedup 1.0000x reference)
import dataclasses
import functools

import jax
import jax.numpy as jnp
from jax.experimental import pallas as pl
from jax.experimental.pallas import tpu as pltpu
from jax.experimental.pallas import tpu_sc as plsc

N = 10000
E = 320000
C = 128
BD = 16
CUTOFF = 5.0
EPS = 1e-8

BN = 1000
BE = 4000
NW = 32
EPW = E // NW
BEK = 40
ZR = 40


def _silu(x):
    return x * jax.nn.sigmoid(x)


def _node_mlp_body(q_ref, w1_ref, b1_ref, w2_ref, b2_ref, x_ref):
    h = _silu(jnp.dot(q_ref[...], w1_ref[...], preferred_element_type=jnp.float32)
              + b1_ref[...])
    x_ref[...] = (jnp.dot(h, w2_ref[...], preferred_element_type=jnp.float32)
                  + b2_ref[...])


def _node_mlp(q, W1, b1, W2, b2):
    return pl.pallas_call(
        _node_mlp_body,
        grid=(N // BN,),
        in_specs=[
            pl.BlockSpec((BN, C), lambda i: (i, 0)),
            pl.BlockSpec((C, C), lambda i: (0, 0)),
            pl.BlockSpec((1, C), lambda i: (0, 0)),
            pl.BlockSpec((C, 3 * C), lambda i: (0, 0)),
            pl.BlockSpec((1, 3 * C), lambda i: (0, 0)),
        ],
        out_specs=pl.BlockSpec((BN, 3 * C), lambda i: (i, 0)),
        out_shape=jax.ShapeDtypeStruct((N, 3 * C), jnp.float32),
    )(q, W1, b1.reshape(1, C), W2, b2.reshape(1, 3 * C))


def _edge_filter_body(ea_ref, ew_ref, wf_ref, bf_ref, wij_ref):
    w = (jnp.dot(ea_ref[...], wf_ref[...], preferred_element_type=jnp.float32)
         + bf_ref[...])
    ew = ew_ref[...]
    fcut = 0.5 * (jnp.cos(jnp.pi * ew / CUTOFF) + 1.0)
    fcut = fcut * (ew < CUTOFF).astype(jnp.float32)
    wij_ref[...] = w * fcut


def _edge_filter(edge_attrs, edge_weights, Wf, bf):
    return pl.pallas_call(
        _edge_filter_body,
        grid=(E // BE,),
        in_specs=[
            pl.BlockSpec((BE, BD), lambda i: (i, 0)),
            pl.BlockSpec((BE, 1), lambda i: (i, 0)),
            pl.BlockSpec((BD, 3 * C), lambda i: (0, 0)),
            pl.BlockSpec((1, 3 * C), lambda i: (0, 0)),
        ],
        out_specs=pl.BlockSpec((BE, 3 * C), lambda i: (i, 0)),
        out_shape=jax.ShapeDtypeStruct((E, 3 * C), jnp.float32),
    )(edge_attrs, edge_weights.reshape(E, 1), Wf, bf.reshape(1, 3 * C))


def _sc_body(x, wij, mu0, mu1, mu2, ii, jj, v0, v1, v2,
             dqp, dmup, m12h,
             acc, jv0, iv0, jv1, iv1, msg, mA, mud, v_vmem,
             semA, semB, semC, semD, semI, semO):
    cid = jax.lax.axis_index("c")
    sid = jax.lax.axis_index("s")
    wid = cid * 16 + sid
    ebase = wid * EPW
    mus = [mu0, mu1, mu2]
    vs = [v0, v1, v2]

    def zero_acc():
        @pl.loop(0, ZR)
        def _(r):
            for cc in range(0, C, 16):
                msg[r, pl.ds(cc, 16)] = jnp.zeros((16,), jnp.float32)

        @pl.loop(0, (N // ZR + 15) // 16)
        def _(k):
            idx = k * 16 + sid

            @pl.when(idx < N // ZR)
            def _():
                pltpu.sync_copy(msg, acc.at[pl.ds(idx * ZR, ZR)])
        plsc.subcore_barrier()

    def flush(dst):
        plsc.subcore_barrier()

        @pl.when(sid == 0)
        def _():
            pltpu.sync_copy(acc, dst)
        plsc.subcore_barrier()

    def fetch_idx(base, jdst, idst):
        cj = pltpu.async_copy(jj.at[pl.ds(base, BEK)], jdst, semI)
        ci = pltpu.async_copy(ii.at[pl.ds(base, BEK)], idst, semI)
        return cj, ci

    zero_acc()

    def phase0(xg, wb):
        p0 = fetch_idx(ebase, jv0, iv0)
        p0[0].wait()
        p0[1].wait()

        @pl.loop(0, EPW, step=2 * BEK)
        def _(eo):
            def drain_m12():
                pltpu.make_async_copy(m12h.at[pl.ds(0, BEK), pl.ds(0, C)],
                                      mA, semO).wait()
                pltpu.make_async_copy(m12h.at[pl.ds(0, BEK), pl.ds(C, C)],
                                      mud, semO).wait()

            def block(base, nxt_base, jvk, ivk, jvn, ivn, first):
                c3 = pltpu.async_copy(x.at[jvk], xg, semA)
                c4 = pltpu.async_copy(wij.at[pl.ds(base, BEK)], wb, semB)
                pn = fetch_idx(nxt_base, jvn, ivn)
                c3.wait()
                c4.wait()
                if first:
                    @pl.when(eo > 0)
                    def _():
                        drain_m12()
                else:
                    drain_m12()

                @pl.loop(0, BEK)
                def _(b):
                    for cc in range(0, C, 16):
                        s0 = (b, pl.ds(cc, 16))
                        s1 = (b, pl.ds(C + cc, 16))
                        s2 = (b, pl.ds(2 * C + cc, 16))
                        msg[s0] = wb[s0] * xg[s0]
                        mA[s0] = wb[s1] * xg[s1]
                        mud[s0] = wb[s2] * xg[s2]

                pltpu.async_copy(mA, m12h.at[pl.ds(base, BEK), pl.ds(0, C)],
                                 semO)
                pltpu.async_copy(mud, m12h.at[pl.ds(base, BEK), pl.ds(C, C)],
                                 semO)
                pltpu.sync_copy(msg, acc.at[ivk], add=True)
                pn[0].wait()
                pn[1].wait()

            baseA = ebase + eo
            baseB = baseA + BEK
            nxtA = jnp.minimum(baseA + 2 * BEK, ebase + EPW - BEK)
            block(baseA, baseB, jv0, iv0, jv1, iv1, True)
            block(baseB, nxtA, jv1, iv1, jv0, iv0, False)

        pltpu.make_async_copy(m12h.at[pl.ds(0, BEK), pl.ds(0, C)],
                              mA, semO).wait()
        pltpu.make_async_copy(m12h.at[pl.ds(0, BEK), pl.ds(C, C)],
                              mud, semO).wait()

    pl.run_scoped(phase0,
                  pltpu.VMEM((BEK, 3 * C), jnp.float32),
                  pltpu.VMEM((BEK, 3 * C), jnp.float32))
    flush(dqp.at[cid])

    def dir_phases(msgq, mAq, mudq, v_vmemq):
        sets = [(jv0, iv0, mud, mA, msg, v_vmem, semA, semB),
                (jv1, iv1, mudq, mAq, msgq, v_vmemq, semC, semD)]

        def fire_inputs(d, base, st):
            jvk, _, mudk, mAk, msgk, vvk, sX, sY = st
            cg = pltpu.async_copy(mus[d].at[jvk], mudk, sX)
            c1 = pltpu.async_copy(m12h.at[pl.ds(base, BEK), pl.ds(0, C)],
                                  mAk, sY)
            c2 = pltpu.async_copy(m12h.at[pl.ds(base, BEK), pl.ds(C, C)],
                                  msgk, sY)
            c3 = pltpu.async_copy(vs[d].at[pl.ds(base, BEK)], vvk, sY)
            return cg, c1, c2, c3

        def drain_inputs(d, st):
            _, _, mudk, mAk, msgk, vvk, sX, sY = st
            pltpu.make_async_copy(mus[d].at[pl.ds(0, BEK)], mudk, sX).wait()
            pltpu.make_async_copy(m12h.at[pl.ds(0, BEK), pl.ds(0, C)],
                                  mAk, sY).wait()
            pltpu.make_async_copy(m12h.at[pl.ds(0, BEK), pl.ds(C, C)],
                                  msgk, sY).wait()
            pltpu.make_async_copy(vs[d].at[pl.ds(0, BEK)], vvk, sY).wait()

        def drain_idx(jdst, idst):
            pltpu.make_async_copy(jj.at[pl.ds(0, BEK)], jdst, semI).wait()
            pltpu.make_async_copy(ii.at[pl.ds(0, BEK)], idst, semI).wait()

        def compute_scatter(st):
            _, ivk, mudk, mAk, msgk, vvk, _, _ = st

            @pl.loop(0, BEK)
            def _(b):
                b16 = jax.lax.broadcast(b, (16,))
                vv = plsc.load_gather(vvk, [b16])
                for cc in range(0, C, 16):
                    sl = (b, pl.ds(cc, 16))
                    msgk[sl] = mAk[sl] * vv + msgk[sl] * mudk[sl]

            pltpu.sync_copy(msgk, acc.at[ivk], add=True)

        for d in range(3):
            zero_acc()
            pltpu.sync_copy(jj.at[pl.ds(ebase, BEK)], jv0)
            pltpu.sync_copy(ii.at[pl.ds(ebase, BEK)], iv0)
            fire_inputs(d, ebase, sets[0])
            pltpu.sync_copy(jj.at[pl.ds(ebase + BEK, BEK)], jv1)
            pltpu.sync_copy(ii.at[pl.ds(ebase + BEK, BEK)], iv1)

            @pl.loop(0, EPW, step=2 * BEK)
            def _(eo):
                baseA = ebase + eo
                baseB = baseA + BEK

                @pl.when(eo > 0)
                def _():
                    drain_idx(jv1, iv1)
                fire_inputs(d, baseB, sets[1])
                drain_inputs(d, sets[0])
                compute_scatter(sets[0])

                @pl.when(eo < EPW - 2 * BEK)
                def _():
                    fetch_idx(baseA + 2 * BEK, jv0, iv0)

                @pl.when(eo < EPW - 2 * BEK)
                def _():
                    drain_idx(jv0, iv0)
                    fire_inputs(d, baseA + 2 * BEK, sets[0])
                drain_inputs(d, sets[1])
                compute_scatter(sets[1])

                @pl.when(eo < EPW - 3 * BEK)
                def _():
                    fetch_idx(baseA + 3 * BEK, jv1, iv1)

            flush(dmup.at[d, cid])

    pl.run_scoped(dir_phases,
                  pltpu.VMEM((BEK, C), jnp.float32),
                  pltpu.VMEM((BEK, C), jnp.float32),
                  pltpu.VMEM((BEK, C), jnp.float32),
                  pltpu.VMEM((BEK,), jnp.float32))


def _sc_edge(x, wij, mu0, mu1, mu2, ii, jj, v0, v1, v2):
    mesh = plsc.VectorSubcoreMesh(core_axis_name="c", subcore_axis_name="s")
    f32 = jnp.float32
    cp = pltpu.CompilerParams()
    if "needs_layout_passes" in pltpu.CompilerParams.__dataclass_fields__:
        cp = dataclasses.replace(cp, needs_layout_passes=False)
    run = pl.kernel(
        _sc_body,
        mesh=mesh,
        compiler_params=cp,
        out_type=[
            jax.ShapeDtypeStruct((2, N, C), f32),
            jax.ShapeDtypeStruct((3, 2, N, C), f32),
            jax.ShapeDtypeStruct((E, 2 * C), f32),
        ],
        scratch_types=[
            pltpu.VMEM_SHARED((N, C), f32),
            pltpu.VMEM((BEK,), jnp.int32),
            pltpu.VMEM((BEK,), jnp.int32),
            pltpu.VMEM((BEK,), jnp.int32),
            pltpu.VMEM((BEK,), jnp.int32),
            pltpu.VMEM((BEK, C), f32),
            pltpu.VMEM((BEK, C), f32),
            pltpu.VMEM((BEK, C), f32),
            pltpu.VMEM((BEK,), f32),
            pltpu.SemaphoreType.DMA,
            pltpu.SemaphoreType.DMA,
            pltpu.SemaphoreType.DMA,
            pltpu.SemaphoreType.DMA,
            pltpu.SemaphoreType.DMA,
            pltpu.SemaphoreType.DMA,
        ],
    )
    return run(x, wij, mu0, mu1, mu2, ii, jj, v0, v1, v2)


def _mixing_body(q_ref, mu0_ref, mu1_ref, mu2_ref,
                 dq0_ref, dq1_ref,
                 dm00_ref, dm01_ref, dm10_ref, dm11_ref, dm20_ref, dm21_ref,
                 wmix_ref, wm1_ref, bm1_ref, wm2_ref, bm2_ref,
                 qo_ref, mo0_ref, mo1_ref, mo2_ref):
    qq = q_ref[...] + dq0_ref[...] + dq1_ref[...]
    mu2 = [mu0_ref[...] + dm00_ref[...] + dm01_ref[...],
           mu1_ref[...] + dm10_ref[...] + dm11_ref[...],
           mu2_ref[...] + dm20_ref[...] + dm21_ref[...]]
    wmix = wmix_ref[...]
    mix = [jnp.dot(m, wmix, preferred_element_type=jnp.float32) for m in mu2]
    muV = [m[:, :C] for m in mix]
    muW = [m[:, C:] for m in mix]
    muVn = jnp.sqrt(muV[0] * muV[0] + muV[1] * muV[1] + muV[2] * muV[2] + EPS)
    ctx = jnp.concatenate([qq, muVn], axis=1)
    h = _silu(jnp.dot(ctx, wm1_ref[...], preferred_element_type=jnp.float32)
              + bm1_ref[...])
    y = (jnp.dot(h, wm2_ref[...], preferred_element_type=jnp.float32)
         + bm2_ref[...])
    dq_i = y[:, :C]
    dmu_i = y[:, C:2 * C]
    dqmu_i = y[:, 2 * C:]
    s = muV[0] * muW[0] + muV[1] * muW[1] + muV[2] * muW[2]
    qo_ref[...] = qq + dq_i + dqmu_i * s
    mo0_ref[...] = mu2[0] + dmu_i * muW[0]
    mo1_ref[...] = mu2[1] + dmu_i * muW[1]
    mo2_ref[...] = mu2[2] + dmu_i * muW[2]


def _mixing(q, mu0, mu1, mu2, dq0, dq1, dm00, dm01, dm10, dm11, dm20, dm21,
            Wmix, Wm1, bm1, Wm2, bm2):
    node_spec = pl.BlockSpec((BN, C), lambda i: (i, 0))
    return pl.pallas_call(
        _mixing_body,
        grid=(N // BN,),
        in_specs=[node_spec] * 12 + [
            pl.BlockSpec((C, 2 * C), lambda i: (0, 0)),
            pl.BlockSpec((2 * C, C), lambda i: (0, 0)),
            pl.BlockSpec((1, C), lambda i: (0, 0)),
            pl.BlockSpec((C, 3 * C), lambda i: (0, 0)),
            pl.BlockSpec((1, 3 * C), lambda i: (0, 0)),
        ],
        out_specs=[node_spec, node_spec, node_spec, node_spec],
        out_shape=[jax.ShapeDtypeStruct((N, C), jnp.float32)] * 4,
    )(q, mu0, mu1, mu2, dq0, dq1, dm00, dm01, dm10, dm11, dm20, dm21,
      Wmix, Wm1, bm1.reshape(1, C), Wm2, bm2.reshape(1, 3 * C))


def kernel(q, mu, receivers, edge_indices, edge_weights, edge_versors, edge_attrs,
           W1, b1, W2, b2, Wf, bf, Wmix, Wm1, bm1, Wm2, bm2):
    del receivers
    x = _node_mlp(q, W1, b1, W2, b2)
    wij = _edge_filter(edge_attrs, edge_weights, Wf, bf)

    idx_i = edge_indices[0]
    idx_j = edge_indices[1]
    mu_d = [mu[:, d, :] for d in range(3)]
    v_d = [edge_versors[:, d] for d in range(3)]

    dqp, dmup, _m12 = _sc_edge(x, wij, mu_d[0], mu_d[1], mu_d[2],
                               idx_i, idx_j, v_d[0], v_d[1], v_d[2])

    qo, mo0, mo1, mo2 = _mixing(
        q, mu_d[0], mu_d[1], mu_d[2],
        dqp[0], dqp[1],
        dmup[0, 0], dmup[0, 1], dmup[1, 0], dmup[1, 1], dmup[2, 0], dmup[2, 1],
        Wmix, Wm1, bm1, Wm2, bm2)
    return qo, jnp.stack([mo0, mo1, mo2], axis=1)


# speedup vs baseline: 1.1478x; 1.1478x over previous
import dataclasses
import functools

import jax
import jax.numpy as jnp
from jax.experimental import pallas as pl
from jax.experimental.pallas import tpu as pltpu
from jax.experimental.pallas import tpu_sc as plsc

N = 10000
E = 320000
C = 128
BD = 16
CUTOFF = 5.0
EPS = 1e-8

BN = 1000
BE = 4000
NW = 32
EPW = E // NW
BEK = 40
ZR = 40


def _silu(x):
    return x * jax.nn.sigmoid(x)


def _node_mlp_body(q_ref, w1_ref, b1_ref, w2_ref, b2_ref, x_ref):
    h = _silu(jnp.dot(q_ref[...], w1_ref[...], preferred_element_type=jnp.float32)
              + b1_ref[...])
    x_ref[...] = (jnp.dot(h, w2_ref[...], preferred_element_type=jnp.float32)
                  + b2_ref[...])


def _node_mlp(q, W1, b1, W2, b2):
    return pl.pallas_call(
        _node_mlp_body,
        grid=(N // BN,),
        in_specs=[
            pl.BlockSpec((BN, C), lambda i: (i, 0)),
            pl.BlockSpec((C, C), lambda i: (0, 0)),
            pl.BlockSpec((1, C), lambda i: (0, 0)),
            pl.BlockSpec((C, 3 * C), lambda i: (0, 0)),
            pl.BlockSpec((1, 3 * C), lambda i: (0, 0)),
        ],
        out_specs=pl.BlockSpec((BN, 3 * C), lambda i: (i, 0)),
        out_shape=jax.ShapeDtypeStruct((N, 3 * C), jnp.float32),
    )(q, W1, b1.reshape(1, C), W2, b2.reshape(1, 3 * C))


def _edge_filter_body(ea_ref, ew_ref, wf_ref, bf_ref, wij_ref):
    w = (jnp.dot(ea_ref[...], wf_ref[...], preferred_element_type=jnp.float32)
         + bf_ref[...])
    ew = ew_ref[...]
    fcut = 0.5 * (jnp.cos(jnp.pi * ew / CUTOFF) + 1.0)
    fcut = fcut * (ew < CUTOFF).astype(jnp.float32)
    wij_ref[...] = w * fcut


def _edge_filter(edge_attrs, edge_weights, Wf, bf):
    return pl.pallas_call(
        _edge_filter_body,
        grid=(E // BE,),
        in_specs=[
            pl.BlockSpec((BE, BD), lambda i: (i, 0)),
            pl.BlockSpec((BE, 1), lambda i: (i, 0)),
            pl.BlockSpec((BD, 3 * C), lambda i: (0, 0)),
            pl.BlockSpec((1, 3 * C), lambda i: (0, 0)),
        ],
        out_specs=pl.BlockSpec((BE, 3 * C), lambda i: (i, 0)),
        out_shape=jax.ShapeDtypeStruct((E, 3 * C), jnp.float32),
    )(edge_attrs, edge_weights.reshape(E, 1), Wf, bf.reshape(1, 3 * C))


def _sc_body(x, wij, mu0, mu1, mu2, ii, jj, v0, v1, v2,
             dqp, dmup, m12h,
             acc, jv0, iv0, jv1, iv1, msg, mA, mud, v_vmem,
             semA, semB, semC, semD, semI, semO):
    cid = jax.lax.axis_index("c")
    sid = jax.lax.axis_index("s")
    wid = cid * 16 + sid
    ebase = wid * EPW
    mus = [mu0, mu1, mu2]
    vs = [v0, v1, v2]

    def zero_acc():
        @pl.loop(0, ZR)
        def _(r):
            for cc in range(0, C, 16):
                msg[r, pl.ds(cc, 16)] = jnp.zeros((16,), jnp.float32)

        @pl.loop(0, (N // ZR + 15) // 16)
        def _(k):
            idx = k * 16 + sid

            @pl.when(idx < N // ZR)
            def _():
                pltpu.sync_copy(msg, acc.at[pl.ds(idx * ZR, ZR)])
        plsc.subcore_barrier()

    def flush(dst):
        plsc.subcore_barrier()

        @pl.when(sid == 0)
        def _():
            pltpu.sync_copy(acc, dst)
        plsc.subcore_barrier()

    def fetch_idx(base, jdst, idst):
        cj = pltpu.async_copy(jj.at[pl.ds(base, BEK)], jdst, semI)
        ci = pltpu.async_copy(ii.at[pl.ds(base, BEK)], idst, semI)
        return cj, ci

    zero_acc()

    def phase0(xg, wb):
        p0 = fetch_idx(ebase, jv0, iv0)
        p0[0].wait()
        p0[1].wait()

        @pl.loop(0, EPW, step=2 * BEK)
        def _(eo):
            def block(base, nxt_base, jvk, ivk, jvn, ivn):
                c3 = pltpu.async_copy(x.at[jvk], xg, semA)
                c4 = pltpu.async_copy(wij.at[pl.ds(base, BEK)], wb, semB)
                pn = fetch_idx(nxt_base, jvn, ivn)
                c3.wait()
                c4.wait()

                @pl.loop(0, BEK)
                def _(b):
                    for cc in range(0, C, 16):
                        s0 = (b, pl.ds(cc, 16))
                        s1 = (b, pl.ds(C + cc, 16))
                        s2 = (b, pl.ds(2 * C + cc, 16))
                        msg[s0] = wb[s0] * xg[s0]
                        xg[s1] = wb[s1] * xg[s1]
                        xg[s2] = wb[s2] * xg[s2]

                c5 = pltpu.async_copy(
                    xg.at[pl.ds(0, BEK), pl.ds(C, C)],
                    m12h.at[pl.ds(base, BEK), pl.ds(0, C)], semO)
                c6 = pltpu.async_copy(
                    xg.at[pl.ds(0, BEK), pl.ds(2 * C, C)],
                    m12h.at[pl.ds(base, BEK), pl.ds(C, C)], semO)
                pltpu.sync_copy(msg, acc.at[ivk], add=True)
                pn[0].wait()
                pn[1].wait()
                return c5, c6

            baseA = ebase + eo
            baseB = baseA + BEK
            nxtA = jnp.minimum(baseA + 2 * BEK, ebase + EPW - BEK)
            wA = block(baseA, baseB, jv0, iv0, jv1, iv1)
            wA[0].wait()
            wA[1].wait()
            wB = block(baseB, nxtA, jv1, iv1, jv0, iv0)
            wB[0].wait()
            wB[1].wait()

    pl.run_scoped(phase0,
                  pltpu.VMEM((BEK, 3 * C), jnp.float32),
                  pltpu.VMEM((BEK, 3 * C), jnp.float32))
    flush(dqp.at[cid])

    def dir_phases(msgq, mAq, mudq, v_vmemq):
        sets = [(jv0, iv0, mud, mA, msg, v_vmem, semA, semB),
                (jv1, iv1, mudq, mAq, msgq, v_vmemq, semC, semD)]

        def fire_inputs(d, base, st):
            jvk, _, mudk, mAk, msgk, vvk, sX, sY = st
            cg = pltpu.async_copy(mus[d].at[jvk], mudk, sX)
            c1 = pltpu.async_copy(m12h.at[pl.ds(base, BEK), pl.ds(0, C)],
                                  mAk, sY)
            c2 = pltpu.async_copy(m12h.at[pl.ds(base, BEK), pl.ds(C, C)],
                                  msgk, sY)
            c3 = pltpu.async_copy(vs[d].at[pl.ds(base, BEK)], vvk, sY)
            return cg, c1, c2, c3

        def drain_inputs(d, st):
            _, _, mudk, mAk, msgk, vvk, sX, sY = st
            pltpu.make_async_copy(mus[d].at[pl.ds(0, BEK)], mudk, sX).wait()
            pltpu.make_async_copy(m12h.at[pl.ds(0, BEK), pl.ds(0, C)],
                                  mAk, sY).wait()
            pltpu.make_async_copy(m12h.at[pl.ds(0, BEK), pl.ds(C, C)],
                                  msgk, sY).wait()
            pltpu.make_async_copy(vs[d].at[pl.ds(0, BEK)], vvk, sY).wait()

        def drain_idx(jdst, idst):
            pltpu.make_async_copy(jj.at[pl.ds(0, BEK)], jdst, semI).wait()
            pltpu.make_async_copy(ii.at[pl.ds(0, BEK)], idst, semI).wait()

        def compute_scatter(st):
            _, ivk, mudk, mAk, msgk, vvk, _, _ = st

            @pl.loop(0, BEK)
            def _(b):
                b16 = jax.lax.broadcast(b, (16,))
                vv = plsc.load_gather(vvk, [b16])
                for cc in range(0, C, 16):
                    sl = (b, pl.ds(cc, 16))
                    msgk[sl] = mAk[sl] * vv + msgk[sl] * mudk[sl]

            pltpu.sync_copy(msgk, acc.at[ivk], add=True)

        for d in range(3):
            zero_acc()
            pltpu.sync_copy(jj.at[pl.ds(ebase, BEK)], jv0)
            pltpu.sync_copy(ii.at[pl.ds(ebase, BEK)], iv0)
            fire_inputs(d, ebase, sets[0])
            pltpu.sync_copy(jj.at[pl.ds(ebase + BEK, BEK)], jv1)
            pltpu.sync_copy(ii.at[pl.ds(ebase + BEK, BEK)], iv1)

            @pl.loop(0, EPW, step=2 * BEK)
            def _(eo):
                baseA = ebase + eo
                baseB = baseA + BEK

                @pl.when(eo > 0)
                def _():
                    drain_idx(jv1, iv1)
                fire_inputs(d, baseB, sets[1])
                drain_inputs(d, sets[0])
                compute_scatter(sets[0])

                @pl.when(eo < EPW - 2 * BEK)
                def _():
                    fetch_idx(baseA + 2 * BEK, jv0, iv0)

                @pl.when(eo < EPW - 2 * BEK)
                def _():
                    drain_idx(jv0, iv0)
                    fire_inputs(d, baseA + 2 * BEK, sets[0])
                drain_inputs(d, sets[1])
                compute_scatter(sets[1])

                @pl.when(eo < EPW - 3 * BEK)
                def _():
                    fetch_idx(baseA + 3 * BEK, jv1, iv1)

            flush(dmup.at[d, cid])

    pl.run_scoped(dir_phases,
                  pltpu.VMEM((BEK, C), jnp.float32),
                  pltpu.VMEM((BEK, C), jnp.float32),
                  pltpu.VMEM((BEK, C), jnp.float32),
                  pltpu.VMEM((BEK,), jnp.float32))


def _sc_edge(x, wij, mu0, mu1, mu2, ii, jj, v0, v1, v2):
    mesh = plsc.VectorSubcoreMesh(core_axis_name="c", subcore_axis_name="s")
    f32 = jnp.float32
    cp = pltpu.CompilerParams()
    if "needs_layout_passes" in pltpu.CompilerParams.__dataclass_fields__:
        cp = dataclasses.replace(cp, needs_layout_passes=False)
    run = pl.kernel(
        _sc_body,
        mesh=mesh,
        compiler_params=cp,
        out_type=[
            jax.ShapeDtypeStruct((2, N, C), f32),
            jax.ShapeDtypeStruct((3, 2, N, C), f32),
            jax.ShapeDtypeStruct((E, 2 * C), f32),
        ],
        scratch_types=[
            pltpu.VMEM_SHARED((N, C), f32),
            pltpu.VMEM((BEK,), jnp.int32),
            pltpu.VMEM((BEK,), jnp.int32),
            pltpu.VMEM((BEK,), jnp.int32),
            pltpu.VMEM((BEK,), jnp.int32),
            pltpu.VMEM((BEK, C), f32),
            pltpu.VMEM((BEK, C), f32),
            pltpu.VMEM((BEK, C), f32),
            pltpu.VMEM((BEK,), f32),
            pltpu.SemaphoreType.DMA,
            pltpu.SemaphoreType.DMA,
            pltpu.SemaphoreType.DMA,
            pltpu.SemaphoreType.DMA,
            pltpu.SemaphoreType.DMA,
            pltpu.SemaphoreType.DMA,
        ],
    )
    return run(x, wij, mu0, mu1, mu2, ii, jj, v0, v1, v2)


def _mixing_body(q_ref, mu0_ref, mu1_ref, mu2_ref,
                 dq0_ref, dq1_ref,
                 dm00_ref, dm01_ref, dm10_ref, dm11_ref, dm20_ref, dm21_ref,
                 wmix_ref, wm1_ref, bm1_ref, wm2_ref, bm2_ref,
                 qo_ref, mo0_ref, mo1_ref, mo2_ref):
    qq = q_ref[...] + dq0_ref[...] + dq1_ref[...]
    mu2 = [mu0_ref[...] + dm00_ref[...] + dm01_ref[...],
           mu1_ref[...] + dm10_ref[...] + dm11_ref[...],
           mu2_ref[...] + dm20_ref[...] + dm21_ref[...]]
    wmix = wmix_ref[...]
    mix = [jnp.dot(m, wmix, preferred_element_type=jnp.float32) for m in mu2]
    muV = [m[:, :C] for m in mix]
    muW = [m[:, C:] for m in mix]
    muVn = jnp.sqrt(muV[0] * muV[0] + muV[1] * muV[1] + muV[2] * muV[2] + EPS)
    ctx = jnp.concatenate([qq, muVn], axis=1)
    h = _silu(jnp.dot(ctx, wm1_ref[...], preferred_element_type=jnp.float32)
              + bm1_ref[...])
    y = (jnp.dot(h, wm2_ref[...], preferred_element_type=jnp.float32)
         + bm2_ref[...])
    dq_i = y[:, :C]
    dmu_i = y[:, C:2 * C]
    dqmu_i = y[:, 2 * C:]
    s = muV[0] * muW[0] + muV[1] * muW[1] + muV[2] * muW[2]
    qo_ref[...] = qq + dq_i + dqmu_i * s
    mo0_ref[...] = mu2[0] + dmu_i * muW[0]
    mo1_ref[...] = mu2[1] + dmu_i * muW[1]
    mo2_ref[...] = mu2[2] + dmu_i * muW[2]


def _mixing(q, mu0, mu1, mu2, dq0, dq1, dm00, dm01, dm10, dm11, dm20, dm21,
            Wmix, Wm1, bm1, Wm2, bm2):
    node_spec = pl.BlockSpec((BN, C), lambda i: (i, 0))
    return pl.pallas_call(
        _mixing_body,
        grid=(N // BN,),
        in_specs=[node_spec] * 12 + [
            pl.BlockSpec((C, 2 * C), lambda i: (0, 0)),
            pl.BlockSpec((2 * C, C), lambda i: (0, 0)),
            pl.BlockSpec((1, C), lambda i: (0, 0)),
            pl.BlockSpec((C, 3 * C), lambda i: (0, 0)),
            pl.BlockSpec((1, 3 * C), lambda i: (0, 0)),
        ],
        out_specs=[node_spec, node_spec, node_spec, node_spec],
        out_shape=[jax.ShapeDtypeStruct((N, C), jnp.float32)] * 4,
    )(q, mu0, mu1, mu2, dq0, dq1, dm00, dm01, dm10, dm11, dm20, dm21,
      Wmix, Wm1, bm1.reshape(1, C), Wm2, bm2.reshape(1, 3 * C))


def kernel(q, mu, receivers, edge_indices, edge_weights, edge_versors, edge_attrs,
           W1, b1, W2, b2, Wf, bf, Wmix, Wm1, bm1, Wm2, bm2):
    del receivers
    x = _node_mlp(q, W1, b1, W2, b2)
    wij = _edge_filter(edge_attrs, edge_weights, Wf, bf)

    idx_i = edge_indices[0]
    idx_j = edge_indices[1]
    mu_d = [mu[:, d, :] for d in range(3)]
    v_d = [edge_versors[:, d] for d in range(3)]

    dqp, dmup, _m12 = _sc_edge(x, wij, mu_d[0], mu_d[1], mu_d[2],
                               idx_i, idx_j, v_d[0], v_d[1], v_d[2])

    qo, mo0, mo1, mo2 = _mixing(
        q, mu_d[0], mu_d[1], mu_d[2],
        dqp[0], dqp[1],
        dmup[0, 0], dmup[0, 1], dmup[1, 0], dmup[1, 1], dmup[2, 0], dmup[2, 1],
        Wmix, Wm1, bm1, Wm2, bm2)
    return qo, jnp.stack([mo0, mo1, mo2], axis=1)
